# baseline (speedup 1.0000x reference)
import jax
import jax.numpy as jnp
from jax import lax
from jax.experimental import pallas as pl
from jax.experimental.layout import Layout, with_layout_constraint
from jax.experimental.pallas import tpu as pltpu
from jax.experimental.pallas import tpu_sc as plsc

_NC = 2
_NS = 16
_NW = _NC * _NS
_C = 392
_LANES = 128


def _sc_gather3(tables, indices, chunks_per_table):
    out_types = tuple(
        jax.ShapeDtypeStruct((idx.shape[0], _LANES), tab.dtype)
        for tab, idx in zip(tables, indices)
    )
    mesh = plsc.VectorSubcoreMesh(core_axis_name="c", subcore_axis_name="s")

    @pl.kernel(
        out_type=out_types,
        mesh=mesh,
        scratch_types=[
            pltpu.VMEM((_C,), jnp.int32),
            pltpu.VMEM((_C,), jnp.int32),
            pltpu.VMEM((_C, _LANES), jnp.float32),
            pltpu.VMEM((_C, _LANES), jnp.float32),
            pltpu.SemaphoreType.DMA,
            pltpu.SemaphoreType.DMA,
            pltpu.SemaphoreType.DMA,
            pltpu.SemaphoreType.DMA,
        ],
        compiler_params=pltpu.CompilerParams(use_tc_tiling_on_sc=False),
    )
    def gather_kernel(t0, t1, t2, i0, i1, i2, o0, o1, o2, idx_v0, idx_v1,
                      rows_v0, rows_v1, gsem0, gsem1, wsem0, wsem1):
        idx_v = (idx_v0, idx_v1)
        rows_v = (rows_v0, rows_v1)
        gsem = (gsem0, gsem1)
        wsem = (wsem0, wsem1)
        wid = lax.axis_index("s") * _NC + lax.axis_index("c")

        work = []
        for tab, idxr, outr, nch in zip((t0, t1, t2), (i0, i1, i2),
                                        (o0, o1, o2), chunks_per_table):
            n_rows = outr.shape[0]
            base = wid * (nch * _C)
            for c in range(nch):
                off = jnp.minimum(base + c * _C, n_rows - _C)
                work.append((tab, idxr, outr, off))
        n = len(work)

        def start_gather(w, b):
            tab, idxr, _, off = work[w]
            pltpu.sync_copy(idxr.at[pl.ds(off, _C)], idx_v[b])
            return pltpu.async_copy(tab.at[idx_v[b]], rows_v[b], gsem[b])

        def start_writeback(w, b):
            _, _, outr, off = work[w]
            return pltpu.async_copy(rows_v[b], outr.at[pl.ds(off, _C)],
                                    wsem[b])

        gh = [None] * n
        wh = [None] * n
        gh[0] = start_gather(0, 0)
        for w in range(n):
            b = w % 2
            if w + 1 < n:
                if w >= 1:
                    wh[w - 1].wait()
                gh[w + 1] = start_gather(w + 1, 1 - b)
            gh[w].wait()
            wh[w] = start_writeback(w, b)
        wh[n - 2].wait()
        wh[n - 1].wait()

    return gather_kernel(*tables, *indices)


def kernel(E_user, E_item, E_category, idx_user, idx_item, idx_category):
    dim = E_user.shape[1]
    pad = ((0, 0), (0, _LANES - dim))
    outs = _sc_gather3(
        tuple(jnp.pad(t, pad) for t in (E_user, E_item, E_category)),
        (idx_user.astype(jnp.int32), idx_item.astype(jnp.int32),
         idx_category.astype(jnp.int32)),
        (4, 4, 2),
    )
    padded_rows = Layout((0, 1), ((8, 128),))
    return tuple(
        with_layout_constraint(o[:, :dim], padded_rows) for o in outs
    )


# speedup vs baseline: 1.0699x; 1.0699x over previous
import jax
import jax.numpy as jnp
from jax import lax
from jax.experimental import pallas as pl
from jax.experimental.layout import Layout, with_layout_constraint
from jax.experimental.pallas import tpu as pltpu
from jax.experimental.pallas import tpu_sc as plsc

_NC = 2
_NS = 16
_NW = _NC * _NS
_C = 392
_LANES = 128


def _sc_gather3(tables, indices, chunks_per_table):
    dim = tables[0].shape[1]
    out_types = tuple(
        jax.ShapeDtypeStruct((idx.shape[0], _LANES), tab.dtype)
        for tab, idx in zip(tables, indices)
    )
    mesh = plsc.VectorSubcoreMesh(core_axis_name="c", subcore_axis_name="s")

    @pl.kernel(
        out_type=out_types,
        mesh=mesh,
        scratch_types=[
            pltpu.VMEM((_C,), jnp.int32),
            pltpu.VMEM((_C,), jnp.int32),
            pltpu.VMEM((_C, dim), jnp.float32),
            pltpu.VMEM((_C, dim), jnp.float32),
            pltpu.SemaphoreType.DMA,
            pltpu.SemaphoreType.DMA,
            pltpu.SemaphoreType.DMA,
            pltpu.SemaphoreType.DMA,
        ],
        compiler_params=pltpu.CompilerParams(use_tc_tiling_on_sc=False),
    )
    def gather_kernel(t0, t1, t2, i0, i1, i2, o0, o1, o2, idx_v0, idx_v1,
                      rows_v0, rows_v1, gsem0, gsem1, wsem0, wsem1):
        idx_v = (idx_v0, idx_v1)
        rows_v = (rows_v0, rows_v1)
        gsem = (gsem0, gsem1)
        wsem = (wsem0, wsem1)
        wid = lax.axis_index("s") * _NC + lax.axis_index("c")

        work = []
        for tab, idxr, outr, nch in zip((t0, t1, t2), (i0, i1, i2),
                                        (o0, o1, o2), chunks_per_table):
            n_rows = outr.shape[0]
            base = wid * (nch * _C)
            for c in range(nch):
                off = jnp.minimum(base + c * _C, n_rows - _C)
                work.append((tab, idxr, outr, off))
        n = len(work)

        def start_gather(w, b):
            tab, idxr, _, off = work[w]
            pltpu.sync_copy(idxr.at[pl.ds(off, _C)], idx_v[b])
            return pltpu.async_copy(tab.at[idx_v[b]], rows_v[b], gsem[b])

        def start_writeback(w, b):
            _, _, outr, off = work[w]
            return pltpu.async_copy(
                rows_v[b], outr.at[pl.ds(off, _C), pl.ds(0, dim)], wsem[b])

        gh = [None] * n
        wh = [None] * n
        gh[0] = start_gather(0, 0)
        for w in range(n):
            b = w % 2
            if w + 1 < n:
                if w >= 1:
                    wh[w - 1].wait()
                gh[w + 1] = start_gather(w + 1, 1 - b)
            gh[w].wait()
            wh[w] = start_writeback(w, b)
        wh[n - 2].wait()
        wh[n - 1].wait()

    return gather_kernel(*tables, *indices)


def kernel(E_user, E_item, E_category, idx_user, idx_item, idx_category):
    dim = E_user.shape[1]
    outs = _sc_gather3(
        (E_user, E_item, E_category),
        (idx_user.astype(jnp.int32), idx_item.astype(jnp.int32),
         idx_category.astype(jnp.int32)),
        (4, 4, 2),
    )
    padded_rows = Layout((0, 1), ((8, 128),))
    return tuple(
        with_layout_constraint(o[:, :dim], padded_rows) for o in outs
    )


# speedup vs baseline: 1.1072x; 1.0349x over previous
import jax
import jax.numpy as jnp
from jax import lax
from jax.experimental import pallas as pl
from jax.experimental.layout import Layout, with_layout_constraint
from jax.experimental.pallas import tpu as pltpu
from jax.experimental.pallas import tpu_sc as plsc

_NC = 2
_NS = 16
_NW = _NC * _NS
_C = 392
_LANES = 128


def _sc_gather1(table, idx, nch):
    dim = table.shape[1]
    out_type = jax.ShapeDtypeStruct((idx.shape[0], _LANES), table.dtype)
    mesh = plsc.VectorSubcoreMesh(core_axis_name="c", subcore_axis_name="s")

    @pl.kernel(
        out_type=out_type,
        mesh=mesh,
        scratch_types=[
            pltpu.VMEM((_C,), jnp.int32),
            pltpu.VMEM((_C,), jnp.int32),
            pltpu.VMEM((_C, dim), jnp.float32),
            pltpu.VMEM((_C, dim), jnp.float32),
            pltpu.SemaphoreType.DMA,
            pltpu.SemaphoreType.DMA,
            pltpu.SemaphoreType.DMA,
            pltpu.SemaphoreType.DMA,
        ],
        compiler_params=pltpu.CompilerParams(use_tc_tiling_on_sc=False),
    )
    def gather_kernel(tab, idxr, outr, idx_v0, idx_v1, rows_v0, rows_v1,
                      gsem0, gsem1, wsem0, wsem1):
        idx_v = (idx_v0, idx_v1)
        rows_v = (rows_v0, rows_v1)
        gsem = (gsem0, gsem1)
        wsem = (wsem0, wsem1)
        wid = lax.axis_index("s") * _NC + lax.axis_index("c")

        n_rows = outr.shape[0]
        base = wid * (nch * _C)
        offs = [jnp.minimum(base + c * _C, n_rows - _C) for c in range(nch)]
        n = nch

        def start_gather(w, b):
            pltpu.sync_copy(idxr.at[pl.ds(offs[w], _C)], idx_v[b])
            return pltpu.async_copy(tab.at[idx_v[b]], rows_v[b], gsem[b])

        def start_writeback(w, b):
            return pltpu.async_copy(
                rows_v[b], outr.at[pl.ds(offs[w], _C), pl.ds(0, dim)], wsem[b])

        gh = [None] * n
        wh = [None] * n
        gh[0] = start_gather(0, 0)
        for w in range(n):
            b = w % 2
            if w + 1 < n:
                if w >= 1:
                    wh[w - 1].wait()
                gh[w + 1] = start_gather(w + 1, 1 - b)
            gh[w].wait()
            wh[w] = start_writeback(w, b)
        wh[n - 2].wait()
        wh[n - 1].wait()

    return gather_kernel(table, idx)


def kernel(E_user, E_item, E_category, idx_user, idx_item, idx_category):
    dim = E_user.shape[1]
    outs = (
        _sc_gather1(E_user, idx_user.astype(jnp.int32), 4),
        _sc_gather1(E_item, idx_item.astype(jnp.int32), 4),
        _sc_gather1(E_category, idx_category.astype(jnp.int32), 2),
    )
    padded_rows = Layout((0, 1), ((8, 128),))
    return tuple(
        with_layout_constraint(o[:, :dim], padded_rows) for o in outs
    )
